# baseline (speedup 1.0000x reference)
import functools

import jax
import jax.numpy as jnp
from jax import lax
from jax.experimental import pallas as pl
from jax.experimental.pallas import tpu as pltpu
from jax.experimental.pallas import tpu_sc as plsc

ENTITY_TOTAL = 1000000
RELATION_TOTAL = 1000
EMB = 64
B = 16384

NC = 2
NS = 16
NW = NC * NS
LANES = 16
KREGS = EMB // LANES

PER_W = B // NW
SUB = 128
NCHUNK = PER_W // SUB
GROUPS = SUB // LANES


def _project_and_score(j, e_ref, p_ref, rp_ref, r_ref):
    ev = [e_ref[j, pl.ds(k * LANES, LANES)] for k in range(KREGS)]
    pv = [p_ref[j, pl.ds(k * LANES, LANES)] for k in range(KREGS)]
    rpv = [rp_ref[j, pl.ds(k * LANES, LANES)] for k in range(KREGS)]
    prod = ev[0] * pv[0]
    for k in range(1, KREGS):
        prod = prod + ev[k] * pv[k]
    dot = jnp.sum(prod)
    en = [ev[k] + dot * rpv[k] for k in range(KREGS)]
    for k in range(KREGS):
        e_ref[j, pl.ds(k * LANES, LANES)] = en[k]
    return en


def _side(j, h_ref, t_ref, r_ref, hp_ref, tp_ref, rp_ref):
    hn = _project_and_score(j, h_ref, hp_ref, rp_ref, r_ref)
    tn = _project_and_score(j, t_ref, tp_ref, rp_ref, r_ref)
    rv = [r_ref[j, pl.ds(k * LANES, LANES)] for k in range(KREGS)]
    s = jnp.abs(hn[0] + rv[0] - tn[0])
    for k in range(1, KREGS):
        s = s + jnp.abs(hn[k] + rv[k] - tn[k])
    return jnp.sum(s)


def _body(pos_h_hbm, pos_t_hbm, pos_r_hbm, neg_h_hbm, neg_t_hbm, neg_r_hbm,
          ent_hbm, rel_hbm, entp_hbm, relp_hbm,
          pos_out, neg_out, phe_out, pte_out, nhe_out, nte_out,
          iph, ipt, ipr, inh, int_, inr,
          phe, pte, pre, php, ptp, prp,
          nhe, nte, nre, nhp, ntp, nrp,
          psc, nsc, sem):
    wid = lax.axis_index("s") * NC + lax.axis_index("c")
    base = wid * PER_W
    lane = lax.iota(jnp.int32, LANES)

    def chunk(c, carry):
        off = base + c * SUB
        pltpu.sync_copy(pos_h_hbm.at[pl.ds(off, SUB)], iph)
        pltpu.sync_copy(pos_t_hbm.at[pl.ds(off, SUB)], ipt)
        pltpu.sync_copy(pos_r_hbm.at[pl.ds(off, SUB)], ipr)
        pltpu.sync_copy(neg_h_hbm.at[pl.ds(off, SUB)], inh)
        pltpu.sync_copy(neg_t_hbm.at[pl.ds(off, SUB)], int_)
        pltpu.sync_copy(neg_r_hbm.at[pl.ds(off, SUB)], inr)
        copies = [
            pltpu.async_copy(ent_hbm.at[iph], phe, sem),
            pltpu.async_copy(ent_hbm.at[ipt], pte, sem),
            pltpu.async_copy(rel_hbm.at[ipr], pre, sem),
            pltpu.async_copy(entp_hbm.at[iph], php, sem),
            pltpu.async_copy(entp_hbm.at[ipt], ptp, sem),
            pltpu.async_copy(relp_hbm.at[ipr], prp, sem),
            pltpu.async_copy(ent_hbm.at[inh], nhe, sem),
            pltpu.async_copy(ent_hbm.at[int_], nte, sem),
            pltpu.async_copy(rel_hbm.at[inr], nre, sem),
            pltpu.async_copy(entp_hbm.at[inh], nhp, sem),
            pltpu.async_copy(entp_hbm.at[int_], ntp, sem),
            pltpu.async_copy(relp_hbm.at[inr], nrp, sem),
        ]
        for cp in copies:
            cp.wait()

        def group(g, carry2):
            pacc = jnp.zeros((LANES,), jnp.float32)
            nacc = jnp.zeros((LANES,), jnp.float32)
            for jj in range(LANES):
                j = g * LANES + jj
                ps = _side(j, phe, pte, pre, php, ptp, prp)
                ns = _side(j, nhe, nte, nre, nhp, ntp, nrp)
                m = lane == jj
                pacc = jnp.where(m, ps, pacc)
                nacc = jnp.where(m, ns, nacc)
            psc[pl.ds(g * LANES, LANES)] = pacc
            nsc[pl.ds(g * LANES, LANES)] = nacc
            return carry2

        lax.fori_loop(0, GROUPS, group, 0)

        pltpu.sync_copy(phe, phe_out.at[pl.ds(off, SUB)])
        pltpu.sync_copy(pte, pte_out.at[pl.ds(off, SUB)])
        pltpu.sync_copy(nhe, nhe_out.at[pl.ds(off, SUB)])
        pltpu.sync_copy(nte, nte_out.at[pl.ds(off, SUB)])
        pltpu.sync_copy(psc, pos_out.at[pl.ds(off, SUB)])
        pltpu.sync_copy(nsc, neg_out.at[pl.ds(off, SUB)])
        return carry

    lax.fori_loop(0, NCHUNK, chunk, 0)


@jax.jit
def kernel(pos_h, pos_t, pos_r, neg_h, neg_t, neg_r,
           ent_emb, rel_emb, ent_proj_emb, rel_proj_emb):
    f32 = jnp.float32
    run = pl.kernel(
        _body,
        out_type=(
            jax.ShapeDtypeStruct((B,), f32),
            jax.ShapeDtypeStruct((B,), f32),
            jax.ShapeDtypeStruct((B, EMB), f32),
            jax.ShapeDtypeStruct((B, EMB), f32),
            jax.ShapeDtypeStruct((B, EMB), f32),
            jax.ShapeDtypeStruct((B, EMB), f32),
        ),
        mesh=plsc.VectorSubcoreMesh(
            core_axis_name="c", subcore_axis_name="s",
            num_cores=NC, num_subcores=NS),
        compiler_params=pltpu.CompilerParams(
            needs_layout_passes=False, use_tc_tiling_on_sc=False),
        scratch_types=(
            [pltpu.VMEM((SUB,), jnp.int32)] * 6
            + [pltpu.VMEM((SUB, EMB), f32)] * 12
            + [pltpu.VMEM((SUB,), f32)] * 2
            + [pltpu.SemaphoreType.DMA]
        ),
    )
    return run(pos_h, pos_t, pos_r, neg_h, neg_t, neg_r,
               ent_emb, rel_emb, ent_proj_emb, rel_proj_emb)
